# baseline (speedup 1.0000x reference)
import jax
import jax.numpy as jnp
from jax import lax
from jax.experimental import pallas as pl
from jax.experimental.pallas import tpu as pltpu
from jax.experimental.pallas import tpu_sc as plsc

VOCAB = 2048
DIM = 96
N_IDX = 16384
NC = 1
NS = 16
L = 16
NW = NC * NS
IDX_CHUNK = 128
CHUNKS = N_IDX // NW // IDX_CHUNK
PER_W = IDX_CHUNK * CHUNKS
V_PER_T = VOCAB // NS
V_VECS = V_PER_T // L
D_VECS = DIM // L

def _body(table_hbm, idx_hbm, out_hbm,
          tabrows_v, idx_v, ones_v, hist_v, counts_v,
          acc_v, partials_v,
          hist_sh, partials_sh, tab_sem, idx_sem):
    s = lax.axis_index("s")
    w = s

    tab_cp = pltpu.async_copy(
        table_hbm.at[pl.ds(s * V_PER_T, V_PER_T)], tabrows_v, tab_sem)

    idx_cps = [
        pltpu.async_copy(
            idx_hbm.at[pl.ds(w * PER_W + j * IDX_CHUNK, IDX_CHUNK)],
            idx_v.at[j], idx_sem)
        for j in range(CHUNKS)
    ]

    for i in range(IDX_CHUNK // L):
        ones_v[pl.ds(i * L, L)] = jnp.ones((L,), jnp.float32)
    for i in range(V_PER_T // L):
        hist_v[pl.ds(i * L, L)] = jnp.zeros((L,), jnp.float32)
    pltpu.sync_copy(hist_v, hist_sh.at[pl.ds(s * V_PER_T, V_PER_T)])
    plsc.subcore_barrier()
    for cp in idx_cps:
        cp.wait()

    sc_cps = [
        pltpu.async_copy(ones_v, hist_sh.at[idx_v.at[j]], idx_sem, add=True)
        for j in range(CHUNKS)
    ]
    for cp in sc_cps:
        cp.wait()

    plsc.subcore_barrier()
    pltpu.sync_copy(hist_sh.at[pl.ds(s * V_PER_T, V_PER_T)], counts_v)

    tab_cp.wait()

    def fma_group(g, acc):
        cvec = counts_v[pl.ds(g * L, L)]
        for k in range(L):
            cb = lax.gather(
                cvec, jnp.full((L, 1), k, jnp.int32),
                dimension_numbers=lax.GatherDimensionNumbers(
                    offset_dims=(), collapsed_slice_dims=(0,),
                    start_index_map=(0,)),
                slice_sizes=(1,),
                mode=lax.GatherScatterMode.PROMISE_IN_BOUNDS)
            acc = tuple(acc[d] + cb * tabrows_v[g * L + k, pl.ds(d * L, L)]
                        for d in range(D_VECS))
        return acc

    acc = lax.fori_loop(0, V_VECS, fma_group,
                        (jnp.zeros((L,), jnp.float32),) * D_VECS,
                        unroll=2)
    for d in range(D_VECS):
        acc_v[pl.ds(d * L, L)] = acc[d]
    pltpu.sync_copy(acc_v, partials_sh.at[s])
    plsc.subcore_barrier()

    @pl.when(s == 0)
    def _():
        pltpu.sync_copy(partials_sh, partials_v)

        def tile_sum(tt, acc):
            return tuple(acc[d] + partials_v[tt, pl.ds(d * L, L)]
                         for d in range(D_VECS))

        tot = lax.fori_loop(0, NS, tile_sum,
                            (jnp.zeros((L,), jnp.float32),) * D_VECS)
        for d in range(D_VECS):
            acc_v[pl.ds(d * L, L)] = tot[d] * (1.0 / N_IDX)
        pltpu.sync_copy(acc_v, out_hbm)


@jax.jit
def _run(table, idx):
    mesh = plsc.VectorSubcoreMesh(
        core_axis_name="c", subcore_axis_name="s", num_cores=NC)
    f = pl.kernel(
        _body,
        out_type=jax.ShapeDtypeStruct((DIM,), jnp.float32),
        mesh=mesh,
        compiler_params=pltpu.CompilerParams(
            use_tc_tiling_on_sc=False, skip_device_barrier=True),
        scratch_types=[
            pltpu.VMEM((V_PER_T, DIM), jnp.float32),
            pltpu.VMEM((CHUNKS, IDX_CHUNK), jnp.int32),
            pltpu.VMEM((IDX_CHUNK,), jnp.float32),
            pltpu.VMEM((V_PER_T,), jnp.float32),
            pltpu.VMEM((V_PER_T,), jnp.float32),
            pltpu.VMEM((DIM,), jnp.float32),
            pltpu.VMEM((NS, DIM), jnp.float32),
            pltpu.VMEM_SHARED((VOCAB,), jnp.float32),
            pltpu.VMEM_SHARED((NS, DIM), jnp.float32),
            pltpu.SemaphoreType.DMA,
            pltpu.SemaphoreType.DMA,
        ],
    )
    return f(table, idx)


def kernel(node_table, node_indices):
    return _run(node_table, node_indices.astype(jnp.int32))


# speedup vs baseline: 1.0638x; 1.0638x over previous
import jax
import jax.numpy as jnp
from jax import lax
from jax.experimental import pallas as pl
from jax.experimental.pallas import tpu as pltpu
from jax.experimental.pallas import tpu_sc as plsc

VOCAB = 2048
DIM = 96
N_IDX = 16384
NC = 1
NS = 16
L = 16
NW = NC * NS
IDX_CHUNK = 128
CHUNKS = N_IDX // NW // IDX_CHUNK
PER_W = IDX_CHUNK * CHUNKS
V_PER_T = VOCAB // NS
V_VECS = V_PER_T // L
D_VECS = DIM // L

def _body(table_hbm, idx_hbm, out_hbm,
          tabrows_v, idx_v, ones_v, hist_v, counts_v,
          acc_v, partials_v,
          hist_sh, partials_sh, tab_sem, idx_sem):
    s = lax.axis_index("s")
    w = s

    tab_cp = pltpu.async_copy(
        table_hbm.at[pl.ds(s * V_PER_T, V_PER_T)], tabrows_v, tab_sem)

    idx_cps = [pltpu.async_copy(idx_hbm.at[w], idx_v, idx_sem)]

    for i in range(IDX_CHUNK // L):
        ones_v[pl.ds(i * L, L)] = jnp.ones((L,), jnp.float32)
    for i in range(V_PER_T // L):
        hist_v[pl.ds(i * L, L)] = jnp.zeros((L,), jnp.float32)
    pltpu.sync_copy(hist_v, hist_sh.at[pl.ds(s * V_PER_T, V_PER_T)])
    plsc.subcore_barrier()
    for cp in idx_cps:
        cp.wait()

    sc_cps = [
        pltpu.async_copy(ones_v, hist_sh.at[idx_v.at[j]], idx_sem, add=True)
        for j in range(CHUNKS)
    ]
    for cp in sc_cps:
        cp.wait()

    plsc.subcore_barrier()
    pltpu.sync_copy(hist_sh.at[pl.ds(s * V_PER_T, V_PER_T)], counts_v)

    tab_cp.wait()

    def fma_group(g, acc):
        cvec = counts_v[pl.ds(g * L, L)]
        for k in range(L):
            cb = lax.gather(
                cvec, jnp.full((L, 1), k, jnp.int32),
                dimension_numbers=lax.GatherDimensionNumbers(
                    offset_dims=(), collapsed_slice_dims=(0,),
                    start_index_map=(0,)),
                slice_sizes=(1,),
                mode=lax.GatherScatterMode.PROMISE_IN_BOUNDS)
            acc = tuple(acc[d] + cb * tabrows_v[g * L + k, pl.ds(d * L, L)]
                        for d in range(D_VECS))
        return acc

    acc = lax.fori_loop(0, V_VECS, fma_group,
                        (jnp.zeros((L,), jnp.float32),) * D_VECS)
    for d in range(D_VECS):
        acc_v[pl.ds(d * L, L)] = acc[d]
    pltpu.sync_copy(acc_v, partials_sh.at[s])
    plsc.subcore_barrier()

    @pl.when(s == 0)
    def _():
        pltpu.sync_copy(partials_sh, partials_v)

        def tile_sum(tt, acc):
            return tuple(acc[d] + partials_v[tt, pl.ds(d * L, L)]
                         for d in range(D_VECS))

        tot = lax.fori_loop(0, NS, tile_sum,
                            (jnp.zeros((L,), jnp.float32),) * D_VECS)
        for d in range(D_VECS):
            acc_v[pl.ds(d * L, L)] = tot[d] * (1.0 / N_IDX)
        pltpu.sync_copy(acc_v, out_hbm)


@jax.jit
def _run(table, idx):
    mesh = plsc.VectorSubcoreMesh(
        core_axis_name="c", subcore_axis_name="s", num_cores=NC)
    f = pl.kernel(
        _body,
        out_type=jax.ShapeDtypeStruct((DIM,), jnp.float32),
        mesh=mesh,
        compiler_params=pltpu.CompilerParams(
            use_tc_tiling_on_sc=False, skip_device_barrier=True),
        scratch_types=[
            pltpu.VMEM((V_PER_T, DIM), jnp.float32),
            pltpu.VMEM((CHUNKS, IDX_CHUNK), jnp.int32),
            pltpu.VMEM((IDX_CHUNK,), jnp.float32),
            pltpu.VMEM((V_PER_T,), jnp.float32),
            pltpu.VMEM((V_PER_T,), jnp.float32),
            pltpu.VMEM((DIM,), jnp.float32),
            pltpu.VMEM((NS, DIM), jnp.float32),
            pltpu.VMEM_SHARED((VOCAB,), jnp.float32),
            pltpu.VMEM_SHARED((NS, DIM), jnp.float32),
            pltpu.SemaphoreType.DMA,
            pltpu.SemaphoreType.DMA,
        ],
    )
    return f(table, idx.reshape(NW, CHUNKS, IDX_CHUNK))


def kernel(node_table, node_indices):
    return _run(node_table, node_indices.astype(jnp.int32))


# speedup vs baseline: 1.0679x; 1.0038x over previous
import jax
import jax.numpy as jnp
from jax import lax
from jax.experimental import pallas as pl
from jax.experimental.pallas import tpu as pltpu
from jax.experimental.pallas import tpu_sc as plsc

VOCAB = 2048
DIM = 96
N_IDX = 16384
NC = 1
NS = 16
L = 16
NW = NC * NS
IDX_CHUNK = 128
CHUNKS = N_IDX // NW // IDX_CHUNK
PER_W = IDX_CHUNK * CHUNKS
V_PER_T = VOCAB // NS
V_VECS = V_PER_T // L
D_VECS = DIM // L

def _body(table_hbm, idx_hbm, out_hbm,
          tabrows_v, idx_v, ones_v, hist_v, counts_v, idx96_v,
          acc_v, partials_v,
          hist_sh, partials_sh, tab_sem, idx_sem):
    s = lax.axis_index("s")
    w = s

    tab_cp = pltpu.async_copy(
        table_hbm.at[pl.ds(s * V_PER_T, V_PER_T)], tabrows_v, tab_sem)

    idx_cps = [pltpu.async_copy(idx_hbm.at[w], idx_v, idx_sem)]

    for i in range(IDX_CHUNK // L):
        ones_v[pl.ds(i * L, L)] = jnp.ones((L,), jnp.float32)
    for i in range(V_PER_T // L):
        hist_v[pl.ds(i * L, L)] = jnp.zeros((L,), jnp.float32)
    iot = lax.iota(jnp.int32, L)
    for d in range(D_VECS):
        idx96_v[pl.ds(d * L, L)] = iot + (d * L)
    pltpu.sync_copy(hist_v, hist_sh.at[pl.ds(s * V_PER_T, V_PER_T)])

    @pl.when(s == 0)
    def _():
        pltpu.sync_copy(hist_v.at[pl.ds(0, DIM)], partials_sh)

    plsc.subcore_barrier()
    for cp in idx_cps:
        cp.wait()

    sc_cps = [
        pltpu.async_copy(ones_v, hist_sh.at[idx_v.at[j]], idx_sem, add=True)
        for j in range(CHUNKS)
    ]
    for cp in sc_cps:
        cp.wait()

    plsc.subcore_barrier()
    pltpu.sync_copy(hist_sh.at[pl.ds(s * V_PER_T, V_PER_T)], counts_v)

    tab_cp.wait()

    def fma_group(g, acc):
        cvec = counts_v[pl.ds(g * L, L)]
        for k in range(L):
            cb = lax.gather(
                cvec, jnp.full((L, 1), k, jnp.int32),
                dimension_numbers=lax.GatherDimensionNumbers(
                    offset_dims=(), collapsed_slice_dims=(0,),
                    start_index_map=(0,)),
                slice_sizes=(1,),
                mode=lax.GatherScatterMode.PROMISE_IN_BOUNDS)
            acc = tuple(acc[d] + cb * tabrows_v[g * L + k, pl.ds(d * L, L)]
                        for d in range(D_VECS))
        return acc

    acc = lax.fori_loop(0, V_VECS, fma_group,
                        (jnp.zeros((L,), jnp.float32),) * D_VECS)
    for d in range(D_VECS):
        acc_v[pl.ds(d * L, L)] = acc[d]
    pltpu.sync_copy(acc_v, partials_sh.at[idx96_v], add=True)
    plsc.subcore_barrier()

    @pl.when(s == 0)
    def _():
        pltpu.sync_copy(partials_sh, partials_v)
        for d in range(D_VECS):
            acc_v[pl.ds(d * L, L)] = (
                partials_v[pl.ds(d * L, L)] * (1.0 / N_IDX))
        pltpu.sync_copy(acc_v, out_hbm)


@jax.jit
def _run(table, idx):
    mesh = plsc.VectorSubcoreMesh(
        core_axis_name="c", subcore_axis_name="s", num_cores=NC)
    f = pl.kernel(
        _body,
        out_type=jax.ShapeDtypeStruct((DIM,), jnp.float32),
        mesh=mesh,
        compiler_params=pltpu.CompilerParams(
            use_tc_tiling_on_sc=False, skip_device_barrier=True),
        scratch_types=[
            pltpu.VMEM((V_PER_T, DIM), jnp.float32),
            pltpu.VMEM((CHUNKS, IDX_CHUNK), jnp.int32),
            pltpu.VMEM((IDX_CHUNK,), jnp.float32),
            pltpu.VMEM((V_PER_T,), jnp.float32),
            pltpu.VMEM((V_PER_T,), jnp.float32),
            pltpu.VMEM((DIM,), jnp.int32),
            pltpu.VMEM((DIM,), jnp.float32),
            pltpu.VMEM((DIM,), jnp.float32),
            pltpu.VMEM_SHARED((VOCAB,), jnp.float32),
            pltpu.VMEM_SHARED((DIM,), jnp.float32),
            pltpu.SemaphoreType.DMA,
            pltpu.SemaphoreType.DMA,
        ],
    )
    return f(table, idx.reshape(NW, CHUNKS, IDX_CHUNK))


def kernel(node_table, node_indices):
    return _run(node_table, node_indices.astype(jnp.int32))
